# Initial kernel scaffold; baseline (speedup 1.0000x reference)
import jax
import jax.numpy as jnp
from jax.experimental import pallas as pl


def kernel(users, pos_items, neg_items, user_embed, item_embed, adj_rows, adj_cols, adj_vals):
    raise NotImplementedError("write your pallas kernel here")


# speedup vs baseline: 1.5057x; 1.5057x over previous
import functools

import jax
import jax.numpy as jnp
from jax import lax
from jax.experimental import pallas as pl
from jax.experimental.pallas import tpu as pltpu
from jax.experimental.pallas import tpu_sc as plsc

N_USERS = 25000
N_NODES = 50000
D = 64
NNZ = 800000
BATCH = 16384

NC = 2
NS = 16
L = 16

HALF = N_NODES // NC
SP_ROWS = HALF + 8
TRASH = HALF
ZBIG = 1568
ZSML = 1560
EK = 80
E_PER_SUB = NNZ // NS
N_ECHUNK = E_PER_SUB // EK

BK = 128
B_PER_W = BATCH // (NC * NS)
N_BCHUNK = B_PER_W // BK

_mesh = plsc.VectorSubcoreMesh(core_axis_name="c", subcore_axis_name="s")


def _spmm_body(t_in, cols, rows, vals, zeros, t_out,
               spmem, idx_c, idx_r, idx_s, vals_v, gbuf, sem):
    c = lax.axis_index("c")
    s = lax.axis_index("s")
    base_row = c * HALF

    zstart = jnp.where(s < 5, s * ZBIG, 5 * ZBIG + (s - 5) * ZSML)

    @pl.when(s < 5)
    def _():
        pltpu.sync_copy(zeros.at[pl.ds(0, ZBIG)], spmem.at[pl.ds(zstart, ZBIG)])

    @pl.when((s >= 5) & (s < NS - 1))
    def _():
        pltpu.sync_copy(zeros.at[pl.ds(0, ZSML)], spmem.at[pl.ds(zstart, ZSML)])

    @pl.when(s == NS - 1)
    def _():
        pltpu.sync_copy(zeros.at[pl.ds(0, ZBIG)], spmem.at[pl.ds(zstart, ZBIG)])

    plsc.subcore_barrier()

    def edge_step(i, carry):
        off = s * E_PER_SUB + i * EK
        pltpu.sync_copy(cols.at[pl.ds(off, EK)], idx_c)
        pltpu.sync_copy(rows.at[pl.ds(off, EK)], idx_r)
        pltpu.sync_copy(vals.at[pl.ds(off, EK)], vals_v)
        pltpu.async_copy(t_in.at[idx_c], gbuf, sem).wait()
        for g in range(EK // L):
            r = idx_r[pl.ds(g * L, L)]
            lr = r - base_row
            ok = (lr >= 0) & (lr < HALF)
            idx_s[pl.ds(g * L, L)] = jnp.where(ok, lr, TRASH)

        def scale_group(g, carry2):
            vvec = vals_v[pl.ds(g * L, L)]
            for e in range(L):
                row = g * L + e
                v = vvec[e]
                for g2 in range(D // L):
                    sl = pl.ds(g2 * L, L)
                    gbuf[row, sl] = gbuf[row, sl] * v
            return carry2

        lax.fori_loop(0, EK // L, scale_group, 0)
        pltpu.sync_copy(gbuf, spmem.at[idx_s], add=True)
        return carry

    lax.fori_loop(0, N_ECHUNK, edge_step, 0)
    plsc.subcore_barrier()

    fstart = jnp.where(s < 5, s * ZBIG, 5 * ZBIG + (s - 5) * ZSML)
    grow = base_row + fstart

    @pl.when(s < 5)
    def _():
        pltpu.sync_copy(spmem.at[pl.ds(fstart, ZBIG)], t_out.at[pl.ds(grow, ZBIG)])

    @pl.when(s >= 5)
    def _():
        pltpu.sync_copy(spmem.at[pl.ds(fstart, ZSML)], t_out.at[pl.ds(grow, ZSML)])


_spmm = pl.kernel(
    _spmm_body,
    out_type=jax.ShapeDtypeStruct((N_NODES, D), jnp.float32),
    mesh=_mesh,
    compiler_params=pltpu.CompilerParams(use_tc_tiling_on_sc=False, needs_layout_passes=False),
    scratch_types=[
        pltpu.VMEM_SHARED((SP_ROWS, D), jnp.float32),
        pltpu.VMEM((EK,), jnp.int32),
        pltpu.VMEM((EK,), jnp.int32),
        pltpu.VMEM((EK,), jnp.int32),
        pltpu.VMEM((EK,), jnp.float32),
        pltpu.VMEM((EK, D), jnp.float32),
        pltpu.SemaphoreType.DMA,
    ],
)


def _batch_body(t0, t1, t2, t3, users, pos, neg, ps_out, ns_out, sq_out,
                iu, ip, iq, bu, bp, bn, tmp, psb, nsb, sqb, sem):
    c = lax.axis_index("c")
    s = lax.axis_index("s")
    wid = s * NC + c
    tables = (t0, t1, t2, t3)

    def chunk_step(t, carry):
        off = wid * B_PER_W + t * BK
        pltpu.sync_copy(users.at[pl.ds(off, BK)], iu)
        pltpu.sync_copy(pos.at[pl.ds(off, BK)], ip)
        pltpu.sync_copy(neg.at[pl.ds(off, BK)], iq)
        for g in range(BK // L):
            sl = pl.ds(g * L, L)
            ip[sl] = ip[sl] + N_USERS
            iq[sl] = iq[sl] + N_USERS

        pltpu.async_copy(t0.at[iu], bu, sem).wait()
        pltpu.async_copy(t0.at[ip], bp, sem).wait()
        pltpu.async_copy(t0.at[iq], bn, sem).wait()

        def sq_group(g, carry2):
            def sq_elem(e, vec):
                row = g * L + e
                acc = (bu[row, pl.ds(0, L)] * bu[row, pl.ds(0, L)]
                       + bp[row, pl.ds(0, L)] * bp[row, pl.ds(0, L)]
                       + bn[row, pl.ds(0, L)] * bn[row, pl.ds(0, L)])
                for g2 in range(1, D // L):
                    sl = pl.ds(g2 * L, L)
                    acc = (acc + bu[row, sl] * bu[row, sl]
                           + bp[row, sl] * bp[row, sl]
                           + bn[row, sl] * bn[row, sl])
                v = jnp.sum(acc)
                return jnp.where(lax.iota(jnp.int32, L) == e, v, vec)

            vec = lax.fori_loop(0, L, sq_elem, jnp.zeros((L,), jnp.float32))
            sqb[pl.ds(g * L, L)] = vec
            return carry2

        lax.fori_loop(0, BK // L, sq_group, 0)

        for k in range(1, 4):
            for idx, acc in ((iu, bu), (ip, bp), (iq, bn)):
                pltpu.async_copy(tables[k].at[idx], tmp, sem).wait()

                def add_row(j, carry2, acc=acc):
                    for g in range(D // L):
                        sl = pl.ds(g * L, L)
                        acc[j, sl] = acc[j, sl] + tmp[j, sl]
                    return carry2

                lax.fori_loop(0, BK, add_row, 0)

        def score_group(g, carry2):
            def score_elem(e, vecs):
                pv, nv = vecs
                row = g * L + e
                u0 = bu[row, pl.ds(0, L)]
                accp = u0 * bp[row, pl.ds(0, L)]
                accn = u0 * bn[row, pl.ds(0, L)]
                for g2 in range(1, D // L):
                    sl = pl.ds(g2 * L, L)
                    uv = bu[row, sl]
                    accp = accp + uv * bp[row, sl]
                    accn = accn + uv * bn[row, sl]
                pe = jnp.sum(accp)
                ne = jnp.sum(accn)
                lane = lax.iota(jnp.int32, L) == e
                return (jnp.where(lane, pe, pv), jnp.where(lane, ne, nv))

            z = jnp.zeros((L,), jnp.float32)
            pv, nv = lax.fori_loop(0, L, score_elem, (z, z))
            psb[pl.ds(g * L, L)] = pv * (1.0 / 16.0)
            nsb[pl.ds(g * L, L)] = nv * (1.0 / 16.0)
            return carry2

        lax.fori_loop(0, BK // L, score_group, 0)

        pltpu.sync_copy(psb, ps_out.at[pl.ds(off, BK)])
        pltpu.sync_copy(nsb, ns_out.at[pl.ds(off, BK)])
        pltpu.sync_copy(sqb, sq_out.at[pl.ds(off, BK)])
        return carry

    lax.fori_loop(0, N_BCHUNK, chunk_step, 0)


_batch = pl.kernel(
    _batch_body,
    out_type=(
        jax.ShapeDtypeStruct((BATCH,), jnp.float32),
        jax.ShapeDtypeStruct((BATCH,), jnp.float32),
        jax.ShapeDtypeStruct((BATCH,), jnp.float32),
    ),
    mesh=_mesh,
    compiler_params=pltpu.CompilerParams(use_tc_tiling_on_sc=False, needs_layout_passes=False),
    scratch_types=[
        pltpu.VMEM((BK,), jnp.int32),
        pltpu.VMEM((BK,), jnp.int32),
        pltpu.VMEM((BK,), jnp.int32),
        pltpu.VMEM((BK, D), jnp.float32),
        pltpu.VMEM((BK, D), jnp.float32),
        pltpu.VMEM((BK, D), jnp.float32),
        pltpu.VMEM((BK, D), jnp.float32),
        pltpu.VMEM((BK,), jnp.float32),
        pltpu.VMEM((BK,), jnp.float32),
        pltpu.VMEM((BK,), jnp.float32),
        pltpu.SemaphoreType.DMA,
    ],
)


def _loss_body(ps_ref, ns_ref, sq_ref, loss_ref, reg_ref):
    d = ps_ref[...] - ns_ref[...]
    sig = 1.0 / (1.0 + jnp.exp(-d))
    loss = -jnp.sum(jnp.log(sig + 1e-08)) * (1.0 / BATCH)
    reg = jnp.sum(sq_ref[...]) * (1.0 / BATCH)
    loss_ref[...] = jnp.full((1, 1), loss, jnp.float32)
    reg_ref[...] = jnp.full((1, 1), reg, jnp.float32)


_loss_tc = pl.pallas_call(
    _loss_body,
    out_shape=(
        jax.ShapeDtypeStruct((1, 1), jnp.float32),
        jax.ShapeDtypeStruct((1, 1), jnp.float32),
    ),
)


@jax.jit
def kernel(users, pos_items, neg_items, user_embed, item_embed,
           adj_rows, adj_cols, adj_vals):
    users = users.astype(jnp.int32)
    pos_items = pos_items.astype(jnp.int32)
    neg_items = neg_items.astype(jnp.int32)
    adj_rows = adj_rows.astype(jnp.int32)
    adj_cols = adj_cols.astype(jnp.int32)

    t0 = jnp.concatenate([user_embed, item_embed], axis=0)
    zeros = jnp.zeros((ZBIG, D), jnp.float32)

    t1 = _spmm(t0, adj_cols, adj_rows, adj_vals, zeros)
    t2 = _spmm(t1, adj_cols, adj_rows, adj_vals, zeros)
    t3 = _spmm(t2, adj_cols, adj_rows, adj_vals, zeros)

    ps, ns, sq = _batch(t0, t1, t2, t3, users, pos_items, neg_items)

    loss, reg = _loss_tc(ps.reshape(128, 128), ns.reshape(128, 128),
                         sq.reshape(128, 128))
    return (loss[0, 0], reg[0, 0])


# speedup vs baseline: 3.1366x; 2.0832x over previous
import functools

import jax
import jax.numpy as jnp
from jax import lax
from jax.experimental import pallas as pl
from jax.experimental.pallas import tpu as pltpu
from jax.experimental.pallas import tpu_sc as plsc

N_USERS = 25000
N_NODES = 50000
D = 64
NNZ = 800000
BATCH = 16384

NC = 2
NS = 16
L = 16

HALF = N_NODES // NC
SP_ROWS = HALF + 8
TRASH = HALF
ZBIG = 1568
ZSML = 1560
EK = 80
BLK = 2000
SUB = BLK // EK
E_PER_SUB = NNZ // NS
N_EBLK = E_PER_SUB // BLK

BK = 128
B_PER_W = BATCH // (NC * NS)
N_BCHUNK = B_PER_W // BK

_mesh = plsc.VectorSubcoreMesh(core_axis_name="c", subcore_axis_name="s")


def _spmm_body(t_in, cols, rows, vals, zeros, t_out,
               spmem, cols_b, rows_b, vals_b, idx_s, gbuf, sem):
    c = lax.axis_index("c")
    s = lax.axis_index("s")
    base_row = c * HALF

    zstart = jnp.where(s < 5, s * ZBIG, 5 * ZBIG + (s - 5) * ZSML)

    @pl.when(s < 5)
    def _():
        pltpu.sync_copy(zeros.at[pl.ds(0, ZBIG)], spmem.at[pl.ds(zstart, ZBIG)])

    @pl.when((s >= 5) & (s < NS - 1))
    def _():
        pltpu.sync_copy(zeros.at[pl.ds(0, ZSML)], spmem.at[pl.ds(zstart, ZSML)])

    @pl.when(s == NS - 1)
    def _():
        pltpu.sync_copy(zeros.at[pl.ds(0, ZBIG)], spmem.at[pl.ds(zstart, ZBIG)])

    plsc.subcore_barrier()

    def block_step(b, carry):
        off = s * E_PER_SUB + b * BLK
        pltpu.sync_copy(cols.at[pl.ds(off, BLK)], cols_b)
        pltpu.sync_copy(rows.at[pl.ds(off, BLK)], rows_b)
        pltpu.sync_copy(vals.at[pl.ds(off, BLK)], vals_b)
        pltpu.async_copy(t_in.at[cols_b.at[pl.ds(0, EK)]],
                         gbuf.at[pl.ds(0, EK)], sem)

        def sub_step(j, carry2):
            p = lax.rem(j, 2)
            pbase = p * EK
            pltpu.make_async_copy(t_in.at[cols_b.at[pl.ds(j * EK, EK)]],
                                  gbuf.at[pl.ds(pbase, EK)], sem).wait()

            @pl.when(j < SUB - 1)
            def _():
                pltpu.async_copy(
                    t_in.at[cols_b.at[pl.ds((j + 1) * EK, EK)]],
                    gbuf.at[pl.ds((1 - p) * EK, EK)], sem)

            for g in range(EK // L):
                r = rows_b[pl.ds(j * EK + g * L, L)]
                lr = r - base_row
                ok = (lr >= 0) & (lr < HALF)
                idx_s[pl.ds(g * L, L)] = jnp.where(ok, lr, TRASH)

            def scale_group(g, carry3):
                vvec = vals_b[pl.ds(j * EK + g * L, L)]
                for e in range(L):
                    row = g * L + e
                    v = vvec[e]
                    for g2 in range(D // L):
                        sl = pl.ds(g2 * L, L)
                        gbuf[pbase + row, sl] = gbuf[pbase + row, sl] * v
                return carry3

            lax.fori_loop(0, EK // L, scale_group, 0)
            pltpu.sync_copy(gbuf.at[pl.ds(pbase, EK)], spmem.at[idx_s],
                            add=True)
            return carry2

        lax.fori_loop(0, SUB, sub_step, 0)
        return carry

    lax.fori_loop(0, N_EBLK, block_step, 0)
    plsc.subcore_barrier()

    fstart = jnp.where(s < 5, s * ZBIG, 5 * ZBIG + (s - 5) * ZSML)
    grow = base_row + fstart

    @pl.when(s < 5)
    def _():
        pltpu.sync_copy(spmem.at[pl.ds(fstart, ZBIG)], t_out.at[pl.ds(grow, ZBIG)])

    @pl.when(s >= 5)
    def _():
        pltpu.sync_copy(spmem.at[pl.ds(fstart, ZSML)], t_out.at[pl.ds(grow, ZSML)])


_spmm = pl.kernel(
    _spmm_body,
    out_type=jax.ShapeDtypeStruct((N_NODES, D), jnp.float32),
    mesh=_mesh,
    compiler_params=pltpu.CompilerParams(use_tc_tiling_on_sc=False, needs_layout_passes=False),
    scratch_types=[
        pltpu.VMEM_SHARED((SP_ROWS, D), jnp.float32),
        pltpu.VMEM((BLK,), jnp.int32),
        pltpu.VMEM((BLK,), jnp.int32),
        pltpu.VMEM((BLK,), jnp.float32),
        pltpu.VMEM((EK,), jnp.int32),
        pltpu.VMEM((2 * EK, D), jnp.float32),
        pltpu.SemaphoreType.DMA,
    ],
)


def _batch_body(t0, t1, t2, t3, users, pos, neg, ps_out, ns_out, sq_out,
                iu, ip, iq, bu, bp, bn, tmp, psb, nsb, sqb, sem):
    c = lax.axis_index("c")
    s = lax.axis_index("s")
    wid = s * NC + c
    tables = (t0, t1, t2, t3)

    def chunk_step(t, carry):
        off = wid * B_PER_W + t * BK
        pltpu.sync_copy(users.at[pl.ds(off, BK)], iu)
        pltpu.sync_copy(pos.at[pl.ds(off, BK)], ip)
        pltpu.sync_copy(neg.at[pl.ds(off, BK)], iq)
        for g in range(BK // L):
            sl = pl.ds(g * L, L)
            ip[sl] = ip[sl] + N_USERS
            iq[sl] = iq[sl] + N_USERS

        pltpu.async_copy(t0.at[iu], bu, sem).wait()
        pltpu.async_copy(t0.at[ip], bp, sem).wait()
        pltpu.async_copy(t0.at[iq], bn, sem).wait()

        def sq_group(g, carry2):
            def sq_elem(e, vec):
                row = g * L + e
                acc = (bu[row, pl.ds(0, L)] * bu[row, pl.ds(0, L)]
                       + bp[row, pl.ds(0, L)] * bp[row, pl.ds(0, L)]
                       + bn[row, pl.ds(0, L)] * bn[row, pl.ds(0, L)])
                for g2 in range(1, D // L):
                    sl = pl.ds(g2 * L, L)
                    acc = (acc + bu[row, sl] * bu[row, sl]
                           + bp[row, sl] * bp[row, sl]
                           + bn[row, sl] * bn[row, sl])
                v = jnp.sum(acc)
                return jnp.where(lax.iota(jnp.int32, L) == e, v, vec)

            vec = lax.fori_loop(0, L, sq_elem, jnp.zeros((L,), jnp.float32))
            sqb[pl.ds(g * L, L)] = vec
            return carry2

        lax.fori_loop(0, BK // L, sq_group, 0)

        for k in range(1, 4):
            for idx, acc in ((iu, bu), (ip, bp), (iq, bn)):
                pltpu.async_copy(tables[k].at[idx], tmp, sem).wait()

                def add_row(j, carry2, acc=acc):
                    for g in range(D // L):
                        sl = pl.ds(g * L, L)
                        acc[j, sl] = acc[j, sl] + tmp[j, sl]
                    return carry2

                lax.fori_loop(0, BK, add_row, 0)

        def score_group(g, carry2):
            def score_elem(e, vecs):
                pv, nv = vecs
                row = g * L + e
                u0 = bu[row, pl.ds(0, L)]
                accp = u0 * bp[row, pl.ds(0, L)]
                accn = u0 * bn[row, pl.ds(0, L)]
                for g2 in range(1, D // L):
                    sl = pl.ds(g2 * L, L)
                    uv = bu[row, sl]
                    accp = accp + uv * bp[row, sl]
                    accn = accn + uv * bn[row, sl]
                pe = jnp.sum(accp)
                ne = jnp.sum(accn)
                lane = lax.iota(jnp.int32, L) == e
                return (jnp.where(lane, pe, pv), jnp.where(lane, ne, nv))

            z = jnp.zeros((L,), jnp.float32)
            pv, nv = lax.fori_loop(0, L, score_elem, (z, z))
            psb[pl.ds(g * L, L)] = pv * (1.0 / 16.0)
            nsb[pl.ds(g * L, L)] = nv * (1.0 / 16.0)
            return carry2

        lax.fori_loop(0, BK // L, score_group, 0)

        pltpu.sync_copy(psb, ps_out.at[pl.ds(off, BK)])
        pltpu.sync_copy(nsb, ns_out.at[pl.ds(off, BK)])
        pltpu.sync_copy(sqb, sq_out.at[pl.ds(off, BK)])
        return carry

    lax.fori_loop(0, N_BCHUNK, chunk_step, 0)


_batch = pl.kernel(
    _batch_body,
    out_type=(
        jax.ShapeDtypeStruct((BATCH,), jnp.float32),
        jax.ShapeDtypeStruct((BATCH,), jnp.float32),
        jax.ShapeDtypeStruct((BATCH,), jnp.float32),
    ),
    mesh=_mesh,
    compiler_params=pltpu.CompilerParams(use_tc_tiling_on_sc=False, needs_layout_passes=False),
    scratch_types=[
        pltpu.VMEM((BK,), jnp.int32),
        pltpu.VMEM((BK,), jnp.int32),
        pltpu.VMEM((BK,), jnp.int32),
        pltpu.VMEM((BK, D), jnp.float32),
        pltpu.VMEM((BK, D), jnp.float32),
        pltpu.VMEM((BK, D), jnp.float32),
        pltpu.VMEM((BK, D), jnp.float32),
        pltpu.VMEM((BK,), jnp.float32),
        pltpu.VMEM((BK,), jnp.float32),
        pltpu.VMEM((BK,), jnp.float32),
        pltpu.SemaphoreType.DMA,
    ],
)


def _loss_body(ps_ref, ns_ref, sq_ref, loss_ref, reg_ref):
    d = ps_ref[...] - ns_ref[...]
    sig = 1.0 / (1.0 + jnp.exp(-d))
    loss = -jnp.sum(jnp.log(sig + 1e-08)) * (1.0 / BATCH)
    reg = jnp.sum(sq_ref[...]) * (1.0 / BATCH)
    loss_ref[...] = jnp.full((1, 1), loss, jnp.float32)
    reg_ref[...] = jnp.full((1, 1), reg, jnp.float32)


_loss_tc = pl.pallas_call(
    _loss_body,
    out_shape=(
        jax.ShapeDtypeStruct((1, 1), jnp.float32),
        jax.ShapeDtypeStruct((1, 1), jnp.float32),
    ),
)


@jax.jit
def kernel(users, pos_items, neg_items, user_embed, item_embed,
           adj_rows, adj_cols, adj_vals):
    users = users.astype(jnp.int32)
    pos_items = pos_items.astype(jnp.int32)
    neg_items = neg_items.astype(jnp.int32)
    adj_rows = adj_rows.astype(jnp.int32)
    adj_cols = adj_cols.astype(jnp.int32)

    t0 = jnp.concatenate([user_embed, item_embed], axis=0)
    zeros = jnp.zeros((ZBIG, D), jnp.float32)

    t1 = _spmm(t0, adj_cols, adj_rows, adj_vals, zeros)
    t2 = _spmm(t1, adj_cols, adj_rows, adj_vals, zeros)
    t3 = _spmm(t2, adj_cols, adj_rows, adj_vals, zeros)

    ps, ns, sq = _batch(t0, t1, t2, t3, users, pos_items, neg_items)

    loss, reg = _loss_tc(ps.reshape(128, 128), ns.reshape(128, 128),
                         sq.reshape(128, 128))
    return (loss[0, 0], reg[0, 0])


# speedup vs baseline: 5.4061x; 1.7235x over previous
import functools

import jax
import jax.numpy as jnp
from jax import lax
from jax.experimental import pallas as pl
from jax.experimental.pallas import tpu as pltpu
from jax.experimental.pallas import tpu_sc as plsc

N_USERS = 25000
N_NODES = 50000
D = 64
NNZ = 800000
BATCH = 16384

NC = 2
NS = 16
L = 16

HALF = N_NODES // NC
SP_ROWS = HALF + 8
TRASH = HALF
ZBIG = 1568
ZSML = 1560
EK = 80
BLK = 2000
SUB = BLK // EK
E_PER_SUB = NNZ // NS
N_EBLK = E_PER_SUB // BLK

BK = 128
B_PER_W = BATCH // (NC * NS)
N_BCHUNK = B_PER_W // BK

_mesh = plsc.VectorSubcoreMesh(core_axis_name="c", subcore_axis_name="s")


def _spmm_body(t_in, cols, rows, vals, zeros, t_out,
               spmem, cols_b, rows_b, vals_b, idx_s, gbuf, sem, ssem):
    c = lax.axis_index("c")
    s = lax.axis_index("s")
    base_row = c * HALF

    zstart = jnp.where(s < 5, s * ZBIG, 5 * ZBIG + (s - 5) * ZSML)

    @pl.when(s < 5)
    def _():
        pltpu.sync_copy(zeros.at[pl.ds(0, ZBIG)], spmem.at[pl.ds(zstart, ZBIG)])

    @pl.when((s >= 5) & (s < NS - 1))
    def _():
        pltpu.sync_copy(zeros.at[pl.ds(0, ZSML)], spmem.at[pl.ds(zstart, ZSML)])

    @pl.when(s == NS - 1)
    def _():
        pltpu.sync_copy(zeros.at[pl.ds(0, ZBIG)], spmem.at[pl.ds(zstart, ZBIG)])

    plsc.subcore_barrier()

    def block_step(b, carry):
        off = s * E_PER_SUB + b * BLK
        pltpu.sync_copy(cols.at[pl.ds(off, BLK)], cols_b)
        pltpu.sync_copy(rows.at[pl.ds(off, BLK)], rows_b)
        pltpu.sync_copy(vals.at[pl.ds(off, BLK)], vals_b)
        pltpu.async_copy(t_in.at[cols_b.at[pl.ds(0, EK)]],
                         gbuf.at[pl.ds(0, EK)], sem)

        def sub_step(j, carry2):
            p = lax.rem(j, 2)
            pbase = p * EK
            qbase = (1 - p) * EK
            pltpu.make_async_copy(t_in.at[cols_b.at[pl.ds(j * EK, EK)]],
                                  gbuf.at[pl.ds(pbase, EK)], sem).wait()

            @pl.when(j > 0)
            def _():
                pltpu.make_async_copy(gbuf.at[pl.ds(qbase, EK)],
                                      spmem.at[pl.ds(0, EK)], ssem).wait()

            @pl.when(j < SUB - 1)
            def _():
                pltpu.async_copy(
                    t_in.at[cols_b.at[pl.ds((j + 1) * EK, EK)]],
                    gbuf.at[pl.ds(qbase, EK)], sem)

            for g in range(EK // L):
                r = rows_b[pl.ds(j * EK + g * L, L)]
                lr = r - base_row
                ok = (lr >= 0) & (lr < HALF)
                idx_s[p, pl.ds(g * L, L)] = jnp.where(ok, lr, TRASH)

            for g in range(EK // L):
                vvec = vals_b[pl.ds(j * EK + g * L, L)]
                for e in range(L):
                    row = pbase + g * L + e
                    v = vvec[e]
                    for g2 in range(D // L):
                        sl = pl.ds(g2 * L, L)
                        gbuf[row, sl] = gbuf[row, sl] * v

            pltpu.async_copy(gbuf.at[pl.ds(pbase, EK)],
                            spmem.at[idx_s.at[p]], ssem,
                            add=True)
            return carry2

        lax.fori_loop(0, SUB, sub_step, 0)
        pltpu.make_async_copy(gbuf.at[pl.ds(0, EK)],
                              spmem.at[pl.ds(0, EK)], ssem).wait()
        return carry

    lax.fori_loop(0, N_EBLK, block_step, 0)
    plsc.subcore_barrier()

    fstart = jnp.where(s < 5, s * ZBIG, 5 * ZBIG + (s - 5) * ZSML)
    grow = base_row + fstart

    @pl.when(s < 5)
    def _():
        pltpu.sync_copy(spmem.at[pl.ds(fstart, ZBIG)], t_out.at[pl.ds(grow, ZBIG)])

    @pl.when(s >= 5)
    def _():
        pltpu.sync_copy(spmem.at[pl.ds(fstart, ZSML)], t_out.at[pl.ds(grow, ZSML)])


_spmm = pl.kernel(
    _spmm_body,
    out_type=jax.ShapeDtypeStruct((N_NODES, D), jnp.float32),
    mesh=_mesh,
    compiler_params=pltpu.CompilerParams(use_tc_tiling_on_sc=False, needs_layout_passes=False),
    scratch_types=[
        pltpu.VMEM_SHARED((SP_ROWS, D), jnp.float32),
        pltpu.VMEM((BLK,), jnp.int32),
        pltpu.VMEM((BLK,), jnp.int32),
        pltpu.VMEM((BLK,), jnp.float32),
        pltpu.VMEM((2, EK), jnp.int32),
        pltpu.VMEM((2 * EK, D), jnp.float32),
        pltpu.SemaphoreType.DMA,
        pltpu.SemaphoreType.DMA,
    ],
)


def _batch_body(t0, t1, t2, t3, users, pos, neg, ps_out, ns_out, sq_out,
                iu, ip, iq, bu, bp, bn, tmp, psb, nsb, sqb, sem):
    c = lax.axis_index("c")
    s = lax.axis_index("s")
    wid = s * NC + c
    tables = (t0, t1, t2, t3)

    def chunk_step(t, carry):
        off = wid * B_PER_W + t * BK
        pltpu.sync_copy(users.at[pl.ds(off, BK)], iu)
        pltpu.sync_copy(pos.at[pl.ds(off, BK)], ip)
        pltpu.sync_copy(neg.at[pl.ds(off, BK)], iq)
        for g in range(BK // L):
            sl = pl.ds(g * L, L)
            ip[sl] = ip[sl] + N_USERS
            iq[sl] = iq[sl] + N_USERS

        pltpu.async_copy(t0.at[iu], bu, sem).wait()
        pltpu.async_copy(t0.at[ip], bp, sem).wait()
        pltpu.async_copy(t0.at[iq], bn, sem).wait()

        def sq_group(g, carry2):
            def sq_elem(e, vec):
                row = g * L + e
                acc = (bu[row, pl.ds(0, L)] * bu[row, pl.ds(0, L)]
                       + bp[row, pl.ds(0, L)] * bp[row, pl.ds(0, L)]
                       + bn[row, pl.ds(0, L)] * bn[row, pl.ds(0, L)])
                for g2 in range(1, D // L):
                    sl = pl.ds(g2 * L, L)
                    acc = (acc + bu[row, sl] * bu[row, sl]
                           + bp[row, sl] * bp[row, sl]
                           + bn[row, sl] * bn[row, sl])
                v = jnp.sum(acc)
                return jnp.where(lax.iota(jnp.int32, L) == e, v, vec)

            vec = lax.fori_loop(0, L, sq_elem, jnp.zeros((L,), jnp.float32))
            sqb[pl.ds(g * L, L)] = vec
            return carry2

        lax.fori_loop(0, BK // L, sq_group, 0)

        for k in range(1, 4):
            for idx, acc in ((iu, bu), (ip, bp), (iq, bn)):
                pltpu.async_copy(tables[k].at[idx], tmp, sem).wait()

                def add_row(j, carry2, acc=acc):
                    for g in range(D // L):
                        sl = pl.ds(g * L, L)
                        acc[j, sl] = acc[j, sl] + tmp[j, sl]
                    return carry2

                lax.fori_loop(0, BK, add_row, 0)

        def score_group(g, carry2):
            def score_elem(e, vecs):
                pv, nv = vecs
                row = g * L + e
                u0 = bu[row, pl.ds(0, L)]
                accp = u0 * bp[row, pl.ds(0, L)]
                accn = u0 * bn[row, pl.ds(0, L)]
                for g2 in range(1, D // L):
                    sl = pl.ds(g2 * L, L)
                    uv = bu[row, sl]
                    accp = accp + uv * bp[row, sl]
                    accn = accn + uv * bn[row, sl]
                pe = jnp.sum(accp)
                ne = jnp.sum(accn)
                lane = lax.iota(jnp.int32, L) == e
                return (jnp.where(lane, pe, pv), jnp.where(lane, ne, nv))

            z = jnp.zeros((L,), jnp.float32)
            pv, nv = lax.fori_loop(0, L, score_elem, (z, z))
            psb[pl.ds(g * L, L)] = pv * (1.0 / 16.0)
            nsb[pl.ds(g * L, L)] = nv * (1.0 / 16.0)
            return carry2

        lax.fori_loop(0, BK // L, score_group, 0)

        pltpu.sync_copy(psb, ps_out.at[pl.ds(off, BK)])
        pltpu.sync_copy(nsb, ns_out.at[pl.ds(off, BK)])
        pltpu.sync_copy(sqb, sq_out.at[pl.ds(off, BK)])
        return carry

    lax.fori_loop(0, N_BCHUNK, chunk_step, 0)


_batch = pl.kernel(
    _batch_body,
    out_type=(
        jax.ShapeDtypeStruct((BATCH,), jnp.float32),
        jax.ShapeDtypeStruct((BATCH,), jnp.float32),
        jax.ShapeDtypeStruct((BATCH,), jnp.float32),
    ),
    mesh=_mesh,
    compiler_params=pltpu.CompilerParams(use_tc_tiling_on_sc=False, needs_layout_passes=False),
    scratch_types=[
        pltpu.VMEM((BK,), jnp.int32),
        pltpu.VMEM((BK,), jnp.int32),
        pltpu.VMEM((BK,), jnp.int32),
        pltpu.VMEM((BK, D), jnp.float32),
        pltpu.VMEM((BK, D), jnp.float32),
        pltpu.VMEM((BK, D), jnp.float32),
        pltpu.VMEM((BK, D), jnp.float32),
        pltpu.VMEM((BK,), jnp.float32),
        pltpu.VMEM((BK,), jnp.float32),
        pltpu.VMEM((BK,), jnp.float32),
        pltpu.SemaphoreType.DMA,
    ],
)


def _loss_body(ps_ref, ns_ref, sq_ref, loss_ref, reg_ref):
    d = ps_ref[...] - ns_ref[...]
    sig = 1.0 / (1.0 + jnp.exp(-d))
    loss = -jnp.sum(jnp.log(sig + 1e-08)) * (1.0 / BATCH)
    reg = jnp.sum(sq_ref[...]) * (1.0 / BATCH)
    loss_ref[...] = jnp.full((1, 1), loss, jnp.float32)
    reg_ref[...] = jnp.full((1, 1), reg, jnp.float32)


_loss_tc = pl.pallas_call(
    _loss_body,
    out_shape=(
        jax.ShapeDtypeStruct((1, 1), jnp.float32),
        jax.ShapeDtypeStruct((1, 1), jnp.float32),
    ),
)


@jax.jit
def kernel(users, pos_items, neg_items, user_embed, item_embed,
           adj_rows, adj_cols, adj_vals):
    users = users.astype(jnp.int32)
    pos_items = pos_items.astype(jnp.int32)
    neg_items = neg_items.astype(jnp.int32)
    adj_rows = adj_rows.astype(jnp.int32)
    adj_cols = adj_cols.astype(jnp.int32)

    t0 = jnp.concatenate([user_embed, item_embed], axis=0)
    zeros = jnp.zeros((ZBIG, D), jnp.float32)

    t1 = _spmm(t0, adj_cols, adj_rows, adj_vals, zeros)
    t2 = _spmm(t1, adj_cols, adj_rows, adj_vals, zeros)
    t3 = _spmm(t2, adj_cols, adj_rows, adj_vals, zeros)

    ps, ns, sq = _batch(t0, t1, t2, t3, users, pos_items, neg_items)

    loss, reg = _loss_tc(ps.reshape(128, 128), ns.reshape(128, 128),
                         sq.reshape(128, 128))
    return (loss[0, 0], reg[0, 0])


# speedup vs baseline: 9.2232x; 1.7061x over previous
import functools

import jax
import jax.numpy as jnp
from jax import lax
from jax.experimental import pallas as pl
from jax.experimental.pallas import tpu as pltpu
from jax.experimental.pallas import tpu_sc as plsc

N_USERS = 25000
N_NODES = 50000
D = 64
NNZ = 800000
BATCH = 16384

NC = 2
NS = 16
L = 16

HALF = N_NODES // NC
SP_ROWS = HALF + 8
TRASH = HALF
ZBIG = 1568
ZSML = 1560
EK = 80

NW = NC * NS
SEG = NNZ // NW
SEGP = 25024
SEGBUF = 25120
FLUSH_STD = SEGP
FLUSH_LAST = 25104
BLK2 = 2000
SUBB = BLK2 // EK
STAGE = BLK2 + EK
EDGE_LEN = (NW - 1) * SEGP + SEGBUF + STAGE
TRASH_ROW = N_NODES
PB = 5000
PGRP = PB // L

BK = 128
B_PER_W = BATCH // (NC * NS)
N_BCHUNK = B_PER_W // BK

_mesh = plsc.VectorSubcoreMesh(core_axis_name="c", subcore_axis_name="s")


def _part_body(rows, cols, vals, cols_p, rows_p, vals_p, counts,
               rows_sb, cols_sb, vals_sb, rows_ob, cols_ob, vals_ob, cbuf):
    c = lax.axis_index("c")
    s = lax.axis_index("s")
    w = s * NC + c
    seg_in = w * SEG
    iota = lax.iota(jnp.int32, L)

    def do_pass(keep_local, pos0):
        def blk_step(b, pos):
            boff = seg_in + b * PB
            pltpu.sync_copy(rows.at[pl.ds(boff, PB)], rows_sb.at[pl.ds(0, PB)])
            pltpu.sync_copy(cols.at[pl.ds(boff, PB)], cols_sb.at[pl.ds(0, PB)])
            pltpu.sync_copy(vals.at[pl.ds(boff, PB)], vals_sb.at[pl.ds(0, PB)])

            def grp(g, pos2):
                o = g * L
                rv = rows_sb[pl.ds(o, L)]
                cv = cols_sb[pl.ds(o, L)]
                vv = vals_sb[pl.ds(o, L)]
                if keep_local:
                    m = rv < HALF
                else:
                    m = rv >= HALF
                m = m & (iota < (PB - o))
                csum = plsc.cumsum(jnp.where(m, 1, 0))
                idx = csum + (pos2 - 1)
                plsc.store_scatter(rows_ob, [idx], rv, mask=m)
                plsc.store_scatter(cols_ob, [idx], cv, mask=m)
                plsc.store_scatter(vals_ob, [idx], vv, mask=m)
                cnt = plsc.all_reduce_population_count(m)
                return pos2 + cnt[0]

            return lax.fori_loop(0, PGRP + 1, grp, pos)

        return lax.fori_loop(0, SEG // PB, blk_step, pos0)

    trash_r = jnp.full((L,), TRASH_ROW, jnp.int32)
    zero_i = jnp.zeros((L,), jnp.int32)
    zero_f = jnp.zeros((L,), jnp.float32)

    n0 = do_pass(True, 0)
    plsc.store_scatter(rows_ob, [n0 + iota], trash_r)
    plsc.store_scatter(cols_ob, [n0 + iota], zero_i)
    plsc.store_scatter(vals_ob, [n0 + iota], zero_f)
    n0_up = lax.shift_left(lax.shift_right_logical(n0 + 7, 3), 3)
    pos_end = do_pass(False, n0_up)
    for t in range(7):
        tidx = pos_end + t * L + iota
        plsc.store_scatter(rows_ob, [tidx], trash_r)
        plsc.store_scatter(cols_ob, [tidx], zero_i)
        plsc.store_scatter(vals_ob, [tidx], zero_f)

    off_out = w * SEGP

    @pl.when(w < NW - 1)
    def _():
        pltpu.sync_copy(rows_ob.at[pl.ds(0, FLUSH_STD)],
                        rows_p.at[pl.ds(off_out, FLUSH_STD)])
        pltpu.sync_copy(cols_ob.at[pl.ds(0, FLUSH_STD)],
                        cols_p.at[pl.ds(off_out, FLUSH_STD)])
        pltpu.sync_copy(vals_ob.at[pl.ds(0, FLUSH_STD)],
                        vals_p.at[pl.ds(off_out, FLUSH_STD)])

    @pl.when(w == NW - 1)
    def _():
        pltpu.sync_copy(rows_ob.at[pl.ds(0, FLUSH_LAST)],
                        rows_p.at[pl.ds(off_out, FLUSH_LAST)])
        pltpu.sync_copy(cols_ob.at[pl.ds(0, FLUSH_LAST)],
                        cols_p.at[pl.ds(off_out, FLUSH_LAST)])
        pltpu.sync_copy(vals_ob.at[pl.ds(0, FLUSH_LAST)],
                        vals_p.at[pl.ds(off_out, FLUSH_LAST)])

    cbuf[pl.ds(0, L)] = jnp.where(iota == 0, n0, 0)
    pltpu.sync_copy(cbuf.at[pl.ds(0, L)], counts.at[w])


_partition = pl.kernel(
    _part_body,
    out_type=(
        jax.ShapeDtypeStruct((EDGE_LEN,), jnp.int32),
        jax.ShapeDtypeStruct((EDGE_LEN,), jnp.int32),
        jax.ShapeDtypeStruct((EDGE_LEN,), jnp.float32),
        jax.ShapeDtypeStruct((NW, L), jnp.int32),
    ),
    mesh=_mesh,
    compiler_params=pltpu.CompilerParams(use_tc_tiling_on_sc=False,
                                         needs_layout_passes=False),
    scratch_types=[
        pltpu.VMEM((PB + 8,), jnp.int32),
        pltpu.VMEM((PB + 8,), jnp.int32),
        pltpu.VMEM((PB + 8,), jnp.float32),
        pltpu.VMEM((SEGBUF,), jnp.int32),
        pltpu.VMEM((SEGBUF,), jnp.int32),
        pltpu.VMEM((SEGBUF,), jnp.float32),
        pltpu.VMEM((L,), jnp.int32),
    ],
)


def _spmm_body(t_in, cols_p, rows_p, vals_p, counts, zeros, t_out,
               spmem, cols_b, rows_b, vals_b, cnt_v, idx_s, gbuf, sem, ssem):
    c = lax.axis_index("c")
    s = lax.axis_index("s")
    base_row = c * HALF

    zstart = jnp.where(s < 5, s * ZBIG, 5 * ZBIG + (s - 5) * ZSML)

    @pl.when(s < 5)
    def _():
        pltpu.sync_copy(zeros.at[pl.ds(0, ZBIG)], spmem.at[pl.ds(zstart, ZBIG)])

    @pl.when((s >= 5) & (s < NS - 1))
    def _():
        pltpu.sync_copy(zeros.at[pl.ds(0, ZSML)], spmem.at[pl.ds(zstart, ZSML)])

    @pl.when(s == NS - 1)
    def _():
        pltpu.sync_copy(zeros.at[pl.ds(0, ZBIG)], spmem.at[pl.ds(zstart, ZBIG)])

    plsc.subcore_barrier()

    for k in range(2):
        seg = 2 * s + k
        soff = seg * SEGP
        pltpu.sync_copy(counts.at[seg], cnt_v)
        n0 = cnt_v[pl.ds(0, L)][0]
        n0_up = lax.shift_left(lax.shift_right_logical(n0 + 7, 3), 3)
        start = pl.multiple_of(soff + jnp.where(c == 0, 0, n0_up), 8)
        cnt = jnp.where(c == 0, n0_up, SEGP - n0_up)
        nchunk = lax.div(cnt + (EK - 1), EK)
        nblk = lax.div(cnt + (BLK2 - 1), BLK2)

        def blk_step(b, carry):
            boff = pl.multiple_of(start + b * BLK2, 8)
            pltpu.sync_copy(cols_p.at[pl.ds(boff, STAGE)], cols_b)
            pltpu.sync_copy(rows_p.at[pl.ds(boff, STAGE)], rows_b)
            pltpu.sync_copy(vals_p.at[pl.ds(boff, STAGE)], vals_b)
            jlo = b * SUBB
            nsub = jnp.minimum(SUBB, nchunk - jlo)
            pltpu.async_copy(t_in.at[cols_b.at[pl.ds(0, EK)]],
                             gbuf.at[pl.ds(lax.rem(jlo, 2) * EK, EK)], sem)

            def sub_step(jj, carry2):
                j = jlo + jj
                p = lax.rem(j, 2)
                pbase = p * EK
                qbase = (1 - p) * EK
                off = pl.multiple_of(jj * EK, 8)
                pltpu.make_async_copy(t_in.at[cols_b.at[pl.ds(off, EK)]],
                                      gbuf.at[pl.ds(pbase, EK)], sem).wait()

                @pl.when(j > 0)
                def _():
                    pltpu.make_async_copy(gbuf.at[pl.ds(qbase, EK)],
                                          spmem.at[pl.ds(0, EK)],
                                          ssem).wait()

                @pl.when(jj < nsub - 1)
                def _():
                    pltpu.async_copy(
                        t_in.at[cols_b.at[pl.ds(off + EK, EK)]],
                        gbuf.at[pl.ds(qbase, EK)], sem)

                for g in range(EK // L):
                    r = rows_b[pl.ds(off + g * L, L)]
                    lr = r - base_row
                    ok = (lr >= 0) & (lr < HALF)
                    idx_s[p, pl.ds(g * L, L)] = jnp.where(ok, lr, TRASH)

                for g in range(EK // L):
                    vvec = vals_b[pl.ds(off + g * L, L)]
                    for e in range(L):
                        row = pbase + g * L + e
                        v = vvec[e]
                        for g2 in range(D // L):
                            sl = pl.ds(g2 * L, L)
                            gbuf[row, sl] = gbuf[row, sl] * v

                pltpu.async_copy(gbuf.at[pl.ds(pbase, EK)],
                                spmem.at[idx_s.at[p]], ssem,
                                add=True)
                return carry2

            lax.fori_loop(0, nsub, sub_step, 0)
            return carry

        lax.fori_loop(0, nblk, blk_step, 0)

        @pl.when(nchunk > 0)
        def _():
            pltpu.make_async_copy(gbuf.at[pl.ds(0, EK)],
                                  spmem.at[pl.ds(0, EK)], ssem).wait()
    plsc.subcore_barrier()

    fstart = jnp.where(s < 5, s * ZBIG, 5 * ZBIG + (s - 5) * ZSML)
    grow = base_row + fstart

    @pl.when(s < 5)
    def _():
        pltpu.sync_copy(spmem.at[pl.ds(fstart, ZBIG)], t_out.at[pl.ds(grow, ZBIG)])

    @pl.when(s >= 5)
    def _():
        pltpu.sync_copy(spmem.at[pl.ds(fstart, ZSML)], t_out.at[pl.ds(grow, ZSML)])


_spmm = pl.kernel(
    _spmm_body,
    out_type=jax.ShapeDtypeStruct((N_NODES, D), jnp.float32),
    mesh=_mesh,
    compiler_params=pltpu.CompilerParams(use_tc_tiling_on_sc=False, needs_layout_passes=False),
    scratch_types=[
        pltpu.VMEM_SHARED((SP_ROWS, D), jnp.float32),
        pltpu.VMEM((STAGE,), jnp.int32),
        pltpu.VMEM((STAGE,), jnp.int32),
        pltpu.VMEM((STAGE,), jnp.float32),
        pltpu.VMEM((L,), jnp.int32),
        pltpu.VMEM((2, EK), jnp.int32),
        pltpu.VMEM((2 * EK, D), jnp.float32),
        pltpu.SemaphoreType.DMA,
        pltpu.SemaphoreType.DMA,
    ],
)


def _batch_body(t0, t1, t2, t3, users, pos, neg, ps_out, ns_out, sq_out,
                iu, ip, iq, bu, bp, bn, tmp, psb, nsb, sqb, sem):
    c = lax.axis_index("c")
    s = lax.axis_index("s")
    wid = s * NC + c
    tables = (t0, t1, t2, t3)

    def chunk_step(t, carry):
        off = wid * B_PER_W + t * BK
        pltpu.sync_copy(users.at[pl.ds(off, BK)], iu)
        pltpu.sync_copy(pos.at[pl.ds(off, BK)], ip)
        pltpu.sync_copy(neg.at[pl.ds(off, BK)], iq)
        for g in range(BK // L):
            sl = pl.ds(g * L, L)
            ip[sl] = ip[sl] + N_USERS
            iq[sl] = iq[sl] + N_USERS

        pltpu.async_copy(t0.at[iu], bu, sem).wait()
        pltpu.async_copy(t0.at[ip], bp, sem).wait()
        pltpu.async_copy(t0.at[iq], bn, sem).wait()

        def sq_group(g, carry2):
            def sq_elem(e, vec):
                row = g * L + e
                acc = (bu[row, pl.ds(0, L)] * bu[row, pl.ds(0, L)]
                       + bp[row, pl.ds(0, L)] * bp[row, pl.ds(0, L)]
                       + bn[row, pl.ds(0, L)] * bn[row, pl.ds(0, L)])
                for g2 in range(1, D // L):
                    sl = pl.ds(g2 * L, L)
                    acc = (acc + bu[row, sl] * bu[row, sl]
                           + bp[row, sl] * bp[row, sl]
                           + bn[row, sl] * bn[row, sl])
                v = jnp.sum(acc)
                return jnp.where(lax.iota(jnp.int32, L) == e, v, vec)

            vec = lax.fori_loop(0, L, sq_elem, jnp.zeros((L,), jnp.float32))
            sqb[pl.ds(g * L, L)] = vec
            return carry2

        lax.fori_loop(0, BK // L, sq_group, 0)

        for k in range(1, 4):
            for idx, acc in ((iu, bu), (ip, bp), (iq, bn)):
                pltpu.async_copy(tables[k].at[idx], tmp, sem).wait()

                def add_row(j, carry2, acc=acc):
                    for g in range(D // L):
                        sl = pl.ds(g * L, L)
                        acc[j, sl] = acc[j, sl] + tmp[j, sl]
                    return carry2

                lax.fori_loop(0, BK, add_row, 0)

        def score_group(g, carry2):
            def score_elem(e, vecs):
                pv, nv = vecs
                row = g * L + e
                u0 = bu[row, pl.ds(0, L)]
                accp = u0 * bp[row, pl.ds(0, L)]
                accn = u0 * bn[row, pl.ds(0, L)]
                for g2 in range(1, D // L):
                    sl = pl.ds(g2 * L, L)
                    uv = bu[row, sl]
                    accp = accp + uv * bp[row, sl]
                    accn = accn + uv * bn[row, sl]
                pe = jnp.sum(accp)
                ne = jnp.sum(accn)
                lane = lax.iota(jnp.int32, L) == e
                return (jnp.where(lane, pe, pv), jnp.where(lane, ne, nv))

            z = jnp.zeros((L,), jnp.float32)
            pv, nv = lax.fori_loop(0, L, score_elem, (z, z))
            psb[pl.ds(g * L, L)] = pv * (1.0 / 16.0)
            nsb[pl.ds(g * L, L)] = nv * (1.0 / 16.0)
            return carry2

        lax.fori_loop(0, BK // L, score_group, 0)

        pltpu.sync_copy(psb, ps_out.at[pl.ds(off, BK)])
        pltpu.sync_copy(nsb, ns_out.at[pl.ds(off, BK)])
        pltpu.sync_copy(sqb, sq_out.at[pl.ds(off, BK)])
        return carry

    lax.fori_loop(0, N_BCHUNK, chunk_step, 0)


_batch = pl.kernel(
    _batch_body,
    out_type=(
        jax.ShapeDtypeStruct((BATCH,), jnp.float32),
        jax.ShapeDtypeStruct((BATCH,), jnp.float32),
        jax.ShapeDtypeStruct((BATCH,), jnp.float32),
    ),
    mesh=_mesh,
    compiler_params=pltpu.CompilerParams(use_tc_tiling_on_sc=False, needs_layout_passes=False),
    scratch_types=[
        pltpu.VMEM((BK,), jnp.int32),
        pltpu.VMEM((BK,), jnp.int32),
        pltpu.VMEM((BK,), jnp.int32),
        pltpu.VMEM((BK, D), jnp.float32),
        pltpu.VMEM((BK, D), jnp.float32),
        pltpu.VMEM((BK, D), jnp.float32),
        pltpu.VMEM((BK, D), jnp.float32),
        pltpu.VMEM((BK,), jnp.float32),
        pltpu.VMEM((BK,), jnp.float32),
        pltpu.VMEM((BK,), jnp.float32),
        pltpu.SemaphoreType.DMA,
    ],
)


def _loss_body(ps_ref, ns_ref, sq_ref, loss_ref, reg_ref):
    d = ps_ref[...] - ns_ref[...]
    sig = 1.0 / (1.0 + jnp.exp(-d))
    loss = -jnp.sum(jnp.log(sig + 1e-08)) * (1.0 / BATCH)
    reg = jnp.sum(sq_ref[...]) * (1.0 / BATCH)
    loss_ref[...] = jnp.full((1, 1), loss, jnp.float32)
    reg_ref[...] = jnp.full((1, 1), reg, jnp.float32)


_loss_tc = pl.pallas_call(
    _loss_body,
    out_shape=(
        jax.ShapeDtypeStruct((1, 1), jnp.float32),
        jax.ShapeDtypeStruct((1, 1), jnp.float32),
    ),
)


@jax.jit
def kernel(users, pos_items, neg_items, user_embed, item_embed,
           adj_rows, adj_cols, adj_vals):
    users = users.astype(jnp.int32)
    pos_items = pos_items.astype(jnp.int32)
    neg_items = neg_items.astype(jnp.int32)
    adj_rows = adj_rows.astype(jnp.int32)
    adj_cols = adj_cols.astype(jnp.int32)

    t0 = jnp.concatenate([user_embed, item_embed], axis=0)
    zeros = jnp.zeros((ZBIG, D), jnp.float32)

    cols_p, rows_p, vals_p, cnts = _partition(adj_rows, adj_cols, adj_vals)
    t1 = _spmm(t0, cols_p, rows_p, vals_p, cnts, zeros)
    t2 = _spmm(t1, cols_p, rows_p, vals_p, cnts, zeros)
    t3 = _spmm(t2, cols_p, rows_p, vals_p, cnts, zeros)

    ps, ns, sq = _batch(t0, t1, t2, t3, users, pos_items, neg_items)

    loss, reg = _loss_tc(ps.reshape(128, 128), ns.reshape(128, 128),
                         sq.reshape(128, 128))
    return (loss[0, 0], reg[0, 0])
